# baseline (speedup 1.0000x reference)
import functools

import jax
import jax.numpy as jnp
from jax import lax
from jax.experimental import pallas as pl
from jax.experimental.pallas import tpu as pltpu
from jax.experimental.pallas import tpu_sc as plsc

_IN_CH = 128
_N_NODES = 10000
_N_PAD = 10240
_N_BLK = 5120
_N_GRID = _N_PAD // _N_BLK
_N_EDGES = 320000

_NC = 2
_NS = 16
_NW = _NC * _NS
_L = 16
_E_PER = _N_EDGES // _NW


def _tc_body(x_ref, w_ref, b_ref, fw_ref, fb_ref, ap_ref):
    fw1 = fw_ref[:, :_IN_CH]
    fw2 = fw_ref[:, _IN_CH:]
    v = jnp.concatenate([fw1 + fw2, fw2 - fw1], axis=0)
    vpad = jnp.concatenate([v, jnp.zeros((6, _IN_CH), jnp.float32)], axis=0)
    u = jnp.dot(vpad, w_ref[...], preferred_element_type=jnp.float32,
                precision=lax.Precision.HIGHEST)
    ap = lax.dot_general(u, x_ref[...], (((1,), (1,)), ((), ())),
                         preferred_element_type=jnp.float32)
    c = jnp.sum(b_ref[...] * (fw2 + fw2)) + fb_ref[0, 0]
    row = lax.broadcasted_iota(jnp.int32, (2, 1), 0)
    ap_ref[...] = ap[:2, :] + jnp.where(row == 0, c, 0.0)


_tc_stage = pl.pallas_call(
    _tc_body,
    grid=(_N_GRID,),
    in_specs=[
        pl.BlockSpec((_N_BLK, _IN_CH), lambda i: (i, 0)),
        pl.BlockSpec((_IN_CH, _IN_CH), lambda i: (0, 0)),
        pl.BlockSpec((1, _IN_CH), lambda i: (0, 0)),
        pl.BlockSpec((1, 2 * _IN_CH), lambda i: (0, 0)),
        pl.BlockSpec((1, 1), lambda i: (0, 0)),
    ],
    out_specs=pl.BlockSpec((2, _N_BLK), lambda i: (0, i)),
    out_shape=jax.ShapeDtypeStruct((2, _N_PAD), jnp.float32),
)


_WIN = 10240


def _sc_body(apf_hbm, ei_hbm, out_hbm,
             ap_v, ei_v, out_v, sem):
    wid = lax.axis_index("s") * _NC + lax.axis_index("c")
    base = wid * _E_PER
    win = pl.multiple_of((base // 128) * 128, 128)
    off = base - win
    c1 = pltpu.async_copy(apf_hbm, ap_v, sem)
    c2 = pltpu.async_copy(ei_hbm.at[:, pl.ds(win, _WIN)], ei_v, sem)
    c1.wait()
    c2.wait()

    @plsc.parallel_loop(0, _E_PER, step=_L, unroll=2)
    def _(i):
        av = plsc.load_gather(ap_v, [ei_v[0, pl.ds(off + i, _L)]])
        pv = plsc.load_gather(ap_v, [ei_v[1, pl.ds(off + i, _L)] + _N_PAD])
        t = av + pv
        e = jnp.exp(t + t)
        out_v[pl.ds(i, _L)] = 1.0 - 2.0 / (e + 1.0)

    pltpu.sync_copy(out_v, out_hbm.at[pl.ds(base, _E_PER)])


def _make_sc_stage():
    return functools.partial(
        pl.kernel,
        mesh=plsc.VectorSubcoreMesh(core_axis_name="c", subcore_axis_name="s"),
        out_type=jax.ShapeDtypeStruct((_N_EDGES,), jnp.float32),
        compiler_params=pltpu.CompilerParams(needs_layout_passes=False),
        scratch_types=[
            pltpu.VMEM((2 * _N_PAD,), jnp.float32),
            pltpu.VMEM((2, _WIN), jnp.int32),
            pltpu.VMEM((_E_PER,), jnp.float32),
            pltpu.SemaphoreType.DMA,
        ],
    )(_sc_body)


def kernel(x, edge_index, W, b, FW, Fb):
    ei = edge_index.astype(jnp.int32)
    ap = _tc_stage(x, W, b.reshape(1, _IN_CH), FW, Fb.reshape(1, 1))
    return _make_sc_stage()(ap.reshape(2 * _N_PAD), ei)


# speedup vs baseline: 1.0367x; 1.0367x over previous
import functools

import jax
import jax.numpy as jnp
from jax import lax
from jax.experimental import pallas as pl
from jax.experimental.pallas import tpu as pltpu
from jax.experimental.pallas import tpu_sc as plsc

_IN_CH = 128
_N_NODES = 10000
_N_PAD = 10240
_N_BLK = 5120
_N_GRID = _N_PAD // _N_BLK
_N_EDGES = 320000

_NC = 2
_NS = 16
_NW = _NC * _NS
_L = 16
_E_PER = _N_EDGES // _NW


def _tc_body(x_ref, w_ref, b_ref, fw_ref, fb_ref, ap_ref):
    fw1 = fw_ref[:, :_IN_CH]
    fw2 = fw_ref[:, _IN_CH:]
    v = jnp.concatenate([fw1 + fw2, fw2 - fw1], axis=0)
    vpad = jnp.concatenate([v, jnp.zeros((6, _IN_CH), jnp.float32)], axis=0)
    u = jnp.dot(vpad, w_ref[...], preferred_element_type=jnp.float32,
                precision=lax.Precision.HIGHEST)
    ap = lax.dot_general(u, x_ref[...], (((1,), (1,)), ((), ())),
                         preferred_element_type=jnp.float32)
    c = jnp.sum(b_ref[...] * (fw2 + fw2)) + fb_ref[0, 0]
    row = lax.broadcasted_iota(jnp.int32, (2, 1), 0)
    ap_ref[...] = ap[:2, :] + jnp.where(row == 0, c, 0.0)


_tc_stage = pl.pallas_call(
    _tc_body,
    grid=(_N_GRID,),
    in_specs=[
        pl.BlockSpec((_N_BLK, _IN_CH), lambda i: (i, 0)),
        pl.BlockSpec((_IN_CH, _IN_CH), lambda i: (0, 0)),
        pl.BlockSpec((1, _IN_CH), lambda i: (0, 0)),
        pl.BlockSpec((1, 2 * _IN_CH), lambda i: (0, 0)),
        pl.BlockSpec((1, 1), lambda i: (0, 0)),
    ],
    out_specs=pl.BlockSpec((2, _N_BLK), lambda i: (0, i)),
    out_shape=jax.ShapeDtypeStruct((2, _N_PAD), jnp.float32),
)


_WIN = 10240


def _sc_body(ap_hbm, ei_hbm, out_hbm,
             ap_v, ei_v, out_v, sem):
    wid = lax.axis_index("s") * _NC + lax.axis_index("c")
    base = wid * _E_PER
    win = pl.multiple_of((base // 128) * 128, 128)
    off = base - win
    c1 = pltpu.async_copy(ap_hbm, ap_v, sem)
    c2 = pltpu.async_copy(ei_hbm.at[:, pl.ds(win, _WIN)], ei_v, sem)
    c1.wait()
    c2.wait()

    @plsc.parallel_loop(0, _E_PER, step=_L, unroll=4)
    def _(i):
        zero = jnp.zeros((_L,), jnp.int32)
        av = plsc.load_gather(ap_v, [zero, ei_v[0, pl.ds(off + i, _L)]])
        pv = plsc.load_gather(ap_v, [zero + 1, ei_v[1, pl.ds(off + i, _L)]])
        t = av + pv
        e = jnp.exp(t + t)
        out_v[pl.ds(i, _L)] = 1.0 - 2.0 / (e + 1.0)

    pltpu.sync_copy(out_v, out_hbm.at[pl.ds(base, _E_PER)])


def _make_sc_stage():
    return functools.partial(
        pl.kernel,
        mesh=plsc.VectorSubcoreMesh(core_axis_name="c", subcore_axis_name="s"),
        out_type=jax.ShapeDtypeStruct((_N_EDGES,), jnp.float32),
        compiler_params=pltpu.CompilerParams(needs_layout_passes=False),
        scratch_types=[
            pltpu.VMEM((2, _N_PAD), jnp.float32),
            pltpu.VMEM((2, _WIN), jnp.int32),
            pltpu.VMEM((_E_PER,), jnp.float32),
            pltpu.SemaphoreType.DMA,
        ],
    )(_sc_body)


def kernel(x, edge_index, W, b, FW, Fb):
    ei = edge_index.astype(jnp.int32)
    ap = _tc_stage(x, W, b.reshape(1, _IN_CH), FW, Fb.reshape(1, 1))
    return _make_sc_stage()(ap, ei)
